# baseline (speedup 1.0000x reference)
import functools

import jax
import jax.numpy as jnp
from jax import lax
from jax.experimental import pallas as pl
from jax.experimental.pallas import tpu as pltpu
from jax.experimental.pallas import tpu_sc as plsc

TOPK = 512
NC, NS, L = 2, 16, 16
NW = NC * NS
BATCHES = 4
TOKENS = 16384
SLICE = 2048
GROUPS = 2
CAND_CAP = 2048
NEG_INF = float("-inf")


def _score_body(x_ref, w_ref, o_ref):
    s = jnp.dot(x_ref[...], w_ref[...])
    o_ref[...] = s.reshape(o_ref.shape)


def _scores(xf, wt):
    m = xf.shape[0]
    blk = 4096
    return pl.pallas_call(
        _score_body,
        grid=(m // blk,),
        in_specs=[
            pl.BlockSpec((blk, xf.shape[1]), lambda i: (i, 0)),
            pl.BlockSpec((xf.shape[1], 1), lambda i: (0, 0)),
        ],
        out_specs=pl.BlockSpec((blk // 128, 128), lambda i: (i, 0)),
        out_shape=jax.ShapeDtypeStruct((m // 128, 128), jnp.float32),
    )(xf, wt)


_GDN = lax.GatherDimensionNumbers(offset_dims=(), collapsed_slice_dims=(0,),
                                  start_index_map=(0,))


def _dyn_gather(v, idx):
    return lax.gather(v, idx[:, None], dimension_numbers=_GDN,
                      slice_sizes=(1,),
                      mode=lax.GatherScatterMode.PROMISE_IN_BOUNDS)


def _bcast_lane(v, lane):
    return _dyn_gather(v, jnp.full((L,), lane, jnp.int32))


def _allmax(v):
    iota = lax.iota(jnp.int32, L)
    for sh in (8, 4, 2, 1):
        v = jnp.maximum(v, _dyn_gather(v, (iota + sh) % L))
    return v


def _allsum(v):
    iota = lax.iota(jnp.int32, L)
    for sh in (8, 4, 2, 1):
        v = v + _dyn_gather(v, (iota + sh) % L)
    return v


def _lane0(v, scratch):
    scratch[...] = v
    return scratch[0]


def _select_topk(x, W, b):
    bsz, t, n, e = x.shape
    tn = t * n
    xf = x.reshape(bsz * tn, e)
    wt = W.reshape(e, 1)
    scores = (_scores(xf, wt).reshape(bsz * tn) + b[0])

    mesh = plsc.VectorSubcoreMesh(core_axis_name="c", subcore_axis_name="s")
    rows_per_tile = BATCHES * TOPK // NW

    @functools.partial(
        pl.kernel,
        mesh=mesh,
        out_type=jax.ShapeDtypeStruct((bsz * TOPK, e), jnp.float32),
        scratch_types=dict(
            sl=pltpu.VMEM((SLICE,), jnp.float32),
            cs=pltpu.VMEM((CAND_CAP + L,), jnp.float32),
            ci=pltpu.VMEM((CAND_CAP + L,), jnp.int32),
            st16f=pltpu.VMEM((L,), jnp.float32),
            st16i=pltpu.VMEM((L,), jnp.int32),
            stat8=pltpu.VMEM((8 * L,), jnp.float32),
            stat8i=pltpu.VMEM((8 * L,), jnp.int32),
            z128=pltpu.VMEM((128,), jnp.int32),
            rt16=pltpu.VMEM((L,), jnp.int32),
            idxv=pltpu.VMEM((rows_per_tile,), jnp.int32),
            rows=pltpu.VMEM((rows_per_tile, e), jnp.float32),
            sh_mm=pltpu.VMEM_SHARED((GROUPS * 8 * L,), jnp.float32),
            sh_cnt=pltpu.VMEM_SHARED((GROUPS * 8 * L,), jnp.int32),
            sh_m=pltpu.VMEM_SHARED((GROUPS * 8 * L,), jnp.int32),
            sh_cs=pltpu.VMEM_SHARED((GROUPS * CAND_CAP,), jnp.float32),
            sh_ci=pltpu.VMEM_SHARED((GROUPS * CAND_CAP,), jnp.int32),
            sh_out=pltpu.VMEM_SHARED((GROUPS * 2 * TOPK,), jnp.int32),
            sem=pltpu.SemaphoreType.DMA,
        ),
    )
    def k(scores_hbm, x_hbm, out_hbm, *, sl, cs, ci, st16f,
          st16i, stat8, stat8i, z128, rt16, idxv, rows, sh_mm, sh_cnt, sh_m,
          sh_cs, sh_ci, sh_out, sem):
        c = lax.axis_index("c")
        s_id = lax.axis_index("s")
        g = s_id // 8
        sub = s_id % 8
        batch = c * GROUPS + g
        iota = lax.iota(jnp.int32, L)

        base_tok = batch * TOKENS + sub * SLICE
        pltpu.sync_copy(scores_hbm.at[pl.ds(pl.multiple_of(base_tok, SLICE), SLICE)], sl)

        def mm_body(kk, carry):
            v = sl[pl.ds(kk * L, L)]
            return jnp.minimum(carry[0], v), jnp.maximum(carry[1], v)

        big = jnp.full((L,), jnp.inf, jnp.float32)
        mnv, mxv = lax.fori_loop(0, SLICE // L, mm_body, (big, -big))
        mn_s = -_allmax(-mnv)
        mx_s = _allmax(mxv)
        st16f[...] = jnp.where(iota < 8, mn_s, mx_s)
        pltpu.sync_copy(st16f, sh_mm.at[pl.ds(pl.multiple_of((g * 8 + sub) * L, L), L)])
        plsc.subcore_barrier()
        pltpu.sync_copy(sh_mm.at[pl.ds(pl.multiple_of(g * 8 * L, 8 * L), 8 * L)], stat8)

        def mm_red(kk, carry):
            v = stat8[pl.ds(kk * L, L)]
            return (jnp.minimum(carry[0], v[0]), jnp.maximum(carry[1], v[8]))

        lo, hi = lax.fori_loop(0, 8, mm_red, (jnp.float32(jnp.inf),
                                              jnp.float32(-jnp.inf)))
        lo = jnp.full((L,), lo, jnp.float32)
        hi = jnp.full((L,), hi, jnp.float32)

        def count_ge(edges):
            def body(kk, st):
                acc, e2 = st
                v = sl[pl.ds(kk * L, L)]
                io = lax.iota(jnp.int32, L)
                for r in range(L):
                    vr = _dyn_gather(v, (io + r) % L)
                    acc = acc + jnp.where(vr >= e2, jnp.int32(1), jnp.int32(0))
                return (acc, e2)

            acc, _ = lax.fori_loop(0, SLICE // L, body,
                                   (jnp.zeros((L,), jnp.int32), edges))
            return acc

        rlo, rhi = lo, hi
        for _round in range(1):
            io = lax.iota(jnp.int32, L)
            step = (rhi - rlo) * (1.0 / L)
            edges = rlo + io.astype(jnp.float32) * step
            st16i[...] = count_ge(edges)
            pltpu.sync_copy(
                st16i,
                sh_cnt.at[pl.ds(pl.multiple_of((g * 8 + sub) * L, L), L)])
            plsc.subcore_barrier()
            pltpu.sync_copy(
                sh_cnt.at[pl.ds(pl.multiple_of(g * 8 * L, 8 * L), 8 * L)],
                stat8i)

            def sum_red(kk, acc):
                return acc + stat8i[pl.ds(kk * L, L)]

            tot = lax.fori_loop(0, 8, sum_red, jnp.zeros((L,), jnp.int32))
            fb = _allsum(jnp.where(tot >= TOPK, jnp.int32(1), jnp.int32(0)))
            lstar = (jnp.minimum(fb, L) - 1).astype(jnp.float32)
            new_lo = rlo + lstar * step
            new_hi = jnp.where(fb < L, new_lo + step, rhi)
            plsc.subcore_barrier()
            rlo, rhi = new_lo, new_hi
        tau = rlo

        def comp_body(kk, st):
            off, tau2 = st
            io = lax.iota(jnp.int32, L)
            v = sl[pl.ds(kk * L, L)]
            m = v >= tau2
            one = jnp.where(m, jnp.int32(1), jnp.int32(0))
            ps = one
            for sh in (1, 2, 4, 8):
                shifted = _dyn_gather(ps, (io - sh) & (L - 1))
                ps = ps + jnp.where(io >= sh, shifted, jnp.int32(0))
            pc = ps[15]
            enc = jnp.where(m, (ps - 1) * L + io, jnp.int32(0x7FFF))
            src_ln = jnp.zeros((L,), jnp.int32)
            for r in range(L):
                ge = _dyn_gather(enc, (io + r) & (L - 1))
                hit = (ge >> 4) == io
                src_ln = src_ln + jnp.where(hit, ge & (L - 1), jnp.int32(0))
            offc = jnp.minimum(off, CAND_CAP - L)
            cs[pl.ds(offc, L)] = _dyn_gather(v, src_ln)
            key = sub * SLICE + kk * L + io
            ci[pl.ds(offc, L)] = _dyn_gather(key, src_ln)
            return (off + pc, tau2)

        m_t, _ = lax.fori_loop(0, SLICE // L, comp_body,
                               (jnp.int32(0), tau))
        m_t = jnp.minimum(m_t, CAND_CAP - 2 * L)
        sent_s = jnp.full((L,), NEG_INF, jnp.float32)
        sent_i = jnp.full((L,), jnp.int32(0x3FFFFFFF), jnp.int32)
        cs[pl.ds(m_t, L)] = sent_s
        ci[pl.ds(m_t, L)] = sent_i
        cs[pl.ds(m_t + L, L)] = sent_s
        ci[pl.ds(m_t + L, L)] = sent_i
        m_pad = ((m_t + L - 1) // L) * L
        st16i[...] = jnp.full((L,), m_pad, jnp.int32)
        pltpu.sync_copy(st16i, sh_m.at[pl.ds(pl.multiple_of((g * 8 + sub) * L, L), L)])
        plsc.subcore_barrier()

        pltpu.sync_copy(sh_m.at[pl.ds(pl.multiple_of(g * 8 * L, 8 * L), 8 * L)], stat8i)

        def pref(kk, carry):
            my_off, tot = carry
            cnt = stat8i[pl.ds(kk * L, L)][0]
            return (jnp.where(kk < sub, my_off + cnt, my_off), tot + cnt)

        my_off, m_all = lax.fori_loop(0, 8, pref, (jnp.int32(0),
                                                   jnp.int32(0)))
        m_all = jnp.minimum(m_all, CAND_CAP)
        my_off = jnp.minimum(my_off, CAND_CAP - L)

        def pub_body(kk, _):
            dst_off = pl.multiple_of(g * CAND_CAP + my_off + kk * L, L)
            pltpu.sync_copy(cs.at[pl.ds(kk * L, L)],
                            sh_cs.at[pl.ds(dst_off, L)])
            pltpu.sync_copy(ci.at[pl.ds(kk * L, L)],
                            sh_ci.at[pl.ds(dst_off, L)])
            return 0

        lax.fori_loop(0, m_pad // L, pub_body, 0)
        z128[...] = jnp.zeros((128,), jnp.int32)
        pltpu.sync_copy(z128, sh_out.at[pl.ds(pl.multiple_of(g * 2 * TOPK + sub * 128, 128), 128)])
        plsc.subcore_barrier()

        def fetch_body(kk, _):
            @pl.when(kk * 128 < m_all)
            def _():
                src_off = pl.multiple_of(g * CAND_CAP + kk * 128, 128)
                pltpu.sync_copy(sh_cs.at[pl.ds(src_off, 128)],
                                cs.at[pl.ds(kk * 128, 128)])
                pltpu.sync_copy(sh_ci.at[pl.ds(src_off, 128)],
                                ci.at[pl.ds(kk * 128, 128)])
            return 0

        lax.fori_loop(0, CAND_CAP // 128, fetch_body, 0)
        n_vregs = (m_all + L - 1) // L

        def rank_outer(ci_i, _):
            cv = sub + ci_i * 8
            s_c = cs[pl.ds(cv * L, L)]
            i_c = ci[pl.ds(cv * L, L)]

            def rank_inner(dv, st):
                rgt, ecnt, s_c2 = st
                s_d = cs[pl.ds(dv * L, L)]
                for lq in range(L):
                    bs = _bcast_lane(s_d, lq)
                    rgt = rgt + jnp.where(bs > s_c2, jnp.int32(1),
                                          jnp.int32(0))
                    ecnt = ecnt + jnp.where(bs == s_c2, jnp.int32(1),
                                            jnp.int32(0))
                return (rgt, ecnt, s_c2)

            rgt, ecnt, _ = lax.fori_loop(0, n_vregs, rank_inner,
                                         (jnp.zeros((L,), jnp.int32),
                                          jnp.zeros((L,), jnp.int32), s_c))
            rt16[...] = jnp.zeros((L,), jnp.int32)
            anytie = _allmax(ecnt)

            @pl.when(anytie[0] > 1)
            def _():
                def tie_inner(dv, st):
                    tacc, s_c3, i_c3 = st
                    s_d = cs[pl.ds(dv * L, L)]
                    i_d = ci[pl.ds(dv * L, L)]
                    for lq in range(L):
                        bs = _bcast_lane(s_d, lq)
                        bi = _bcast_lane(i_d, lq)
                        tie = (bs == s_c3) & (bi < i_c3)
                        tacc = tacc + jnp.where(tie, jnp.int32(1),
                                                jnp.int32(0))
                    return (tacc, s_c3, i_c3)

                s_cw = cs[pl.ds(cv * L, L)]
                i_cw = ci[pl.ds(cv * L, L)]
                tacc, _, _ = lax.fori_loop(0, n_vregs, tie_inner,
                                           (jnp.zeros((L,), jnp.int32),
                                            s_cw, i_cw))
                rt16[...] = tacc

            racc = rgt + rt16[...]
            win = racc < TOPK
            dest = g * 2 * TOPK + jnp.where(win, racc,
                                            TOPK + (i_c & (TOPK - 1)))
            st16i[...] = batch * TOKENS + i_c
            pltpu.sync_copy(st16i, sh_out.at[dest], add=True)
            return 0

        n_my = jnp.maximum((n_vregs - sub + 7) // 8, 0)
        lax.fori_loop(0, n_my, rank_outer, 0)
        plsc.subcore_barrier()

        pltpu.sync_copy(
            sh_out.at[pl.ds(pl.multiple_of(
                g * 2 * TOPK + sub * rows_per_tile, rows_per_tile),
                rows_per_tile)], idxv)
        pltpu.async_copy(x_hbm.at[idxv], rows, sem).wait()
        obase = pl.multiple_of(batch * TOPK + sub * rows_per_tile,
                               rows_per_tile)
        pltpu.sync_copy(rows, out_hbm.at[pl.ds(obase, rows_per_tile)])

    return k(scores, xf)


def kernel(x, W, b):
    bsz, t, n, e = x.shape
    out = _select_topk(x, W, b)
    return out.reshape(bsz, TOPK, e)


# speedup vs baseline: 1.1782x; 1.1782x over previous
import functools

import jax
import jax.numpy as jnp
from jax import lax
from jax.experimental import pallas as pl
from jax.experimental.pallas import tpu as pltpu
from jax.experimental.pallas import tpu_sc as plsc

TOPK = 512
NC, NS, L = 2, 16, 16
NW = NC * NS
BATCHES = 4
TOKENS = 16384
SLICE = 2048
GROUPS = 2
CAND_CAP = 2048
NEG_INF = float("-inf")


def _score_body(x_ref, w_ref, b_ref, o_ref):
    s = jnp.dot(x_ref[...], w_ref[...]) + b_ref[0]
    o_ref[...] = s.reshape(o_ref.shape)


def _scores(xf, wt, b):
    m = xf.shape[0]
    blk = 4096
    return pl.pallas_call(
        _score_body,
        grid=(m // blk,),
        in_specs=[
            pl.BlockSpec((blk, xf.shape[1]), lambda i: (i, 0)),
            pl.BlockSpec((xf.shape[1], 1), lambda i: (0, 0)),
            pl.BlockSpec(memory_space=pltpu.SMEM),
        ],
        out_specs=pl.BlockSpec((blk // 128, 128), lambda i: (i, 0)),
        out_shape=jax.ShapeDtypeStruct((m // 128, 128), jnp.float32),
    )(xf, wt, b)


_GDN = lax.GatherDimensionNumbers(offset_dims=(), collapsed_slice_dims=(0,),
                                  start_index_map=(0,))


def _dyn_gather(v, idx):
    return lax.gather(v, idx[:, None], dimension_numbers=_GDN,
                      slice_sizes=(1,),
                      mode=lax.GatherScatterMode.PROMISE_IN_BOUNDS)


def _bcast_lane(v, lane):
    return _dyn_gather(v, jnp.full((L,), lane, jnp.int32))


def _allmax(v):
    iota = lax.iota(jnp.int32, L)
    for sh in (8, 4, 2, 1):
        v = jnp.maximum(v, _dyn_gather(v, (iota + sh) % L))
    return v


def _allsum(v):
    iota = lax.iota(jnp.int32, L)
    for sh in (8, 4, 2, 1):
        v = v + _dyn_gather(v, (iota + sh) % L)
    return v


def _lane0(v, scratch):
    scratch[...] = v
    return scratch[0]


def _select_topk(x, W, b):
    bsz, t, n, e = x.shape
    tn = t * n
    xf = x.reshape(bsz * tn, e)
    wt = W.reshape(e, 1)
    scores = _scores(xf, wt, b).reshape(bsz * tn)

    mesh = plsc.VectorSubcoreMesh(core_axis_name="c", subcore_axis_name="s")
    rows_per_tile = BATCHES * TOPK // NW

    @functools.partial(
        pl.kernel,
        mesh=mesh,
        out_type=jax.ShapeDtypeStruct((bsz * TOPK, e), jnp.float32),
        scratch_types=dict(
            sl=pltpu.VMEM((SLICE,), jnp.float32),
            cs=pltpu.VMEM((CAND_CAP + L,), jnp.float32),
            ci=pltpu.VMEM((CAND_CAP + L,), jnp.int32),
            st16f=pltpu.VMEM((L,), jnp.float32),
            st16i=pltpu.VMEM((L,), jnp.int32),
            stat8=pltpu.VMEM((8 * L,), jnp.float32),
            stat8i=pltpu.VMEM((8 * L,), jnp.int32),
            z128=pltpu.VMEM((128,), jnp.int32),
            rt16=pltpu.VMEM((L,), jnp.int32),
            idxv=pltpu.VMEM((rows_per_tile,), jnp.int32),
            rows=pltpu.VMEM((rows_per_tile, e), jnp.float32),
            sh_mm=pltpu.VMEM_SHARED((GROUPS * 8 * L,), jnp.float32),
            sh_cnt=pltpu.VMEM_SHARED((GROUPS * 8 * L,), jnp.int32),
            sh_m=pltpu.VMEM_SHARED((GROUPS * 8 * L,), jnp.int32),
            sh_cs=pltpu.VMEM_SHARED((GROUPS * CAND_CAP,), jnp.float32),
            sh_ci=pltpu.VMEM_SHARED((GROUPS * CAND_CAP,), jnp.int32),
            sh_out=pltpu.VMEM_SHARED((GROUPS * 2 * TOPK,), jnp.int32),
            sem=pltpu.SemaphoreType.DMA,
        ),
    )
    def k(scores_hbm, x_hbm, out_hbm, *, sl, cs, ci, st16f,
          st16i, stat8, stat8i, z128, rt16, idxv, rows, sh_mm, sh_cnt, sh_m,
          sh_cs, sh_ci, sh_out, sem):
        c = lax.axis_index("c")
        s_id = lax.axis_index("s")
        g = s_id // 8
        sub = s_id % 8
        batch = c * GROUPS + g
        iota = lax.iota(jnp.int32, L)

        base_tok = batch * TOKENS + sub * SLICE
        pltpu.sync_copy(scores_hbm.at[pl.ds(pl.multiple_of(base_tok, SLICE), SLICE)], sl)

        def mm_body(kk, carry):
            v = sl[pl.ds(kk * L, L)]
            return jnp.minimum(carry[0], v), jnp.maximum(carry[1], v)

        big = jnp.full((L,), jnp.inf, jnp.float32)
        mnv, mxv = lax.fori_loop(0, SLICE // L, mm_body, (big, -big))
        mn_s = -_allmax(-mnv)
        mx_s = _allmax(mxv)
        st16f[...] = jnp.where(iota < 8, mn_s, mx_s)
        pltpu.sync_copy(st16f, sh_mm.at[pl.ds(pl.multiple_of((g * 8 + sub) * L, L), L)])
        plsc.subcore_barrier()
        pltpu.sync_copy(sh_mm.at[pl.ds(pl.multiple_of(g * 8 * L, 8 * L), 8 * L)], stat8)

        def mm_red(kk, carry):
            v = stat8[pl.ds(kk * L, L)]
            return (jnp.minimum(carry[0], v[0]), jnp.maximum(carry[1], v[8]))

        lo, hi = lax.fori_loop(0, 8, mm_red, (jnp.float32(jnp.inf),
                                              jnp.float32(-jnp.inf)))
        lo = jnp.full((L,), lo, jnp.float32)
        hi = jnp.full((L,), hi, jnp.float32)

        def count_ge(edges):
            def body(kk, st):
                acc, e2 = st
                v = sl[pl.ds(kk * L, L)]
                io = lax.iota(jnp.int32, L)
                for r in range(L):
                    vr = _dyn_gather(v, (io + r) % L)
                    acc = acc + jnp.where(vr >= e2, jnp.int32(1), jnp.int32(0))
                return (acc, e2)

            acc, _ = lax.fori_loop(0, SLICE // L, body,
                                   (jnp.zeros((L,), jnp.int32), edges))
            return acc

        rlo, rhi = lo, hi
        for _round in range(2):
            io = lax.iota(jnp.int32, L)
            step = (rhi - rlo) * (1.0 / L)
            edges = rlo + io.astype(jnp.float32) * step
            st16i[...] = count_ge(edges)
            pltpu.sync_copy(
                st16i,
                sh_cnt.at[pl.ds(pl.multiple_of((g * 8 + sub) * L, L), L)])
            plsc.subcore_barrier()
            pltpu.sync_copy(
                sh_cnt.at[pl.ds(pl.multiple_of(g * 8 * L, 8 * L), 8 * L)],
                stat8i)

            def sum_red(kk, acc):
                return acc + stat8i[pl.ds(kk * L, L)]

            tot = lax.fori_loop(0, 8, sum_red, jnp.zeros((L,), jnp.int32))
            fb = _allsum(jnp.where(tot >= TOPK, jnp.int32(1), jnp.int32(0)))
            lstar = (jnp.minimum(fb, L) - 1).astype(jnp.float32)
            new_lo = rlo + lstar * step
            new_hi = jnp.where(fb < L, new_lo + step, rhi)
            plsc.subcore_barrier()
            rlo, rhi = new_lo, new_hi
        tau = rlo

        def comp_body(kk, st):
            off, tau2 = st
            io = lax.iota(jnp.int32, L)
            v = sl[pl.ds(kk * L, L)]
            m = v >= tau2
            one = jnp.where(m, jnp.int32(1), jnp.int32(0))
            ps = one
            for sh in (1, 2, 4, 8):
                shifted = _dyn_gather(ps, (io - sh) & (L - 1))
                ps = ps + jnp.where(io >= sh, shifted, jnp.int32(0))
            pc = ps[15]
            enc = jnp.where(m, (ps - 1) * L + io, jnp.int32(0x7FFF))
            src_ln = jnp.zeros((L,), jnp.int32)
            for r in range(L):
                ge = _dyn_gather(enc, (io + r) & (L - 1))
                hit = (ge >> 4) == io
                src_ln = src_ln + jnp.where(hit, ge & (L - 1), jnp.int32(0))
            offc = jnp.minimum(off, CAND_CAP - L)
            cs[pl.ds(offc, L)] = _dyn_gather(v, src_ln)
            key = sub * SLICE + kk * L + io
            ci[pl.ds(offc, L)] = _dyn_gather(key, src_ln)
            return (off + pc, tau2)

        m_t, _ = lax.fori_loop(0, SLICE // L, comp_body,
                               (jnp.int32(0), tau))
        m_t = jnp.minimum(m_t, CAND_CAP - 2 * L)
        sent_s = jnp.full((L,), NEG_INF, jnp.float32)
        sent_i = jnp.full((L,), jnp.int32(0x3FFFFFFF), jnp.int32)
        cs[pl.ds(m_t, L)] = sent_s
        ci[pl.ds(m_t, L)] = sent_i
        cs[pl.ds(m_t + L, L)] = sent_s
        ci[pl.ds(m_t + L, L)] = sent_i
        m_pad = ((m_t + L - 1) // L) * L
        st16i[...] = jnp.full((L,), m_pad, jnp.int32)
        pltpu.sync_copy(st16i, sh_m.at[pl.ds(pl.multiple_of((g * 8 + sub) * L, L), L)])
        plsc.subcore_barrier()

        pltpu.sync_copy(sh_m.at[pl.ds(pl.multiple_of(g * 8 * L, 8 * L), 8 * L)], stat8i)

        def pref(kk, carry):
            my_off, tot = carry
            cnt = stat8i[pl.ds(kk * L, L)][0]
            return (jnp.where(kk < sub, my_off + cnt, my_off), tot + cnt)

        my_off, m_all = lax.fori_loop(0, 8, pref, (jnp.int32(0),
                                                   jnp.int32(0)))
        m_all = jnp.minimum(m_all, CAND_CAP)
        my_off = jnp.minimum(my_off, CAND_CAP - L)

        def pub_body(kk, _):
            dst_off = pl.multiple_of(g * CAND_CAP + my_off + kk * L, L)
            pltpu.sync_copy(cs.at[pl.ds(kk * L, L)],
                            sh_cs.at[pl.ds(dst_off, L)])
            pltpu.sync_copy(ci.at[pl.ds(kk * L, L)],
                            sh_ci.at[pl.ds(dst_off, L)])
            return 0

        lax.fori_loop(0, m_pad // L, pub_body, 0)
        z128[...] = jnp.zeros((128,), jnp.int32)
        pltpu.sync_copy(z128, sh_out.at[pl.ds(pl.multiple_of(g * 2 * TOPK + sub * 128, 128), 128)])
        plsc.subcore_barrier()

        def fetch_body(kk, _):
            @pl.when(kk * 128 < m_all)
            def _():
                src_off = pl.multiple_of(g * CAND_CAP + kk * 128, 128)
                pltpu.sync_copy(sh_cs.at[pl.ds(src_off, 128)],
                                cs.at[pl.ds(kk * 128, 128)])
                pltpu.sync_copy(sh_ci.at[pl.ds(src_off, 128)],
                                ci.at[pl.ds(kk * 128, 128)])
            return 0

        lax.fori_loop(0, CAND_CAP // 128, fetch_body, 0)
        n_vregs = (m_all + L - 1) // L

        def rank_outer(ci_i, _):
            cv = sub + ci_i * 8
            s_c = cs[pl.ds(cv * L, L)]
            i_c = ci[pl.ds(cv * L, L)]

            def rank_inner(dv, st):
                rgt, ecnt, s_c2 = st
                s_d = cs[pl.ds(dv * L, L)]
                for lq in range(L):
                    bs = _bcast_lane(s_d, lq)
                    rgt = rgt + jnp.where(bs > s_c2, jnp.int32(1),
                                          jnp.int32(0))
                    ecnt = ecnt + jnp.where(bs == s_c2, jnp.int32(1),
                                            jnp.int32(0))
                return (rgt, ecnt, s_c2)

            rgt, ecnt, _ = lax.fori_loop(0, n_vregs, rank_inner,
                                         (jnp.zeros((L,), jnp.int32),
                                          jnp.zeros((L,), jnp.int32), s_c))
            rt16[...] = jnp.zeros((L,), jnp.int32)
            anytie = _allmax(ecnt)

            @pl.when(anytie[0] > 1)
            def _():
                def tie_inner(dv, st):
                    tacc, s_c3, i_c3 = st
                    s_d = cs[pl.ds(dv * L, L)]
                    i_d = ci[pl.ds(dv * L, L)]
                    for lq in range(L):
                        bs = _bcast_lane(s_d, lq)
                        bi = _bcast_lane(i_d, lq)
                        tie = (bs == s_c3) & (bi < i_c3)
                        tacc = tacc + jnp.where(tie, jnp.int32(1),
                                                jnp.int32(0))
                    return (tacc, s_c3, i_c3)

                s_cw = cs[pl.ds(cv * L, L)]
                i_cw = ci[pl.ds(cv * L, L)]
                tacc, _, _ = lax.fori_loop(0, n_vregs, tie_inner,
                                           (jnp.zeros((L,), jnp.int32),
                                            s_cw, i_cw))
                rt16[...] = tacc

            racc = rgt + rt16[...]
            win = racc < TOPK
            dest = g * 2 * TOPK + jnp.where(win, racc,
                                            TOPK + (i_c & (TOPK - 1)))
            st16i[...] = batch * TOKENS + i_c
            pltpu.sync_copy(st16i, sh_out.at[dest], add=True)
            return 0

        n_my = jnp.maximum((n_vregs - sub + 7) // 8, 0)
        lax.fori_loop(0, n_my, rank_outer, 0)
        plsc.subcore_barrier()

        pltpu.sync_copy(
            sh_out.at[pl.ds(pl.multiple_of(
                g * 2 * TOPK + sub * rows_per_tile, rows_per_tile),
                rows_per_tile)], idxv)
        pltpu.async_copy(x_hbm.at[idxv], rows, sem).wait()
        obase = pl.multiple_of(batch * TOPK + sub * rows_per_tile,
                               rows_per_tile)
        pltpu.sync_copy(rows, out_hbm.at[pl.ds(obase, rows_per_tile)])

    return k(scores, xf)


def kernel(x, W, b):
    bsz, t, n, e = x.shape
    out = _select_topk(x, W, b)
    return out.reshape(bsz, TOPK, e)
